# baseline (speedup 1.0000x reference)
import jax
import jax.numpy as jnp
from jax.experimental import pallas as pl
from jax.experimental.pallas import tpu as pltpu


def _rnn_update_kernel(feature_smem, idxs_vmem, wih_t, whh_t, bias, h0_ref,
                       table_in, table_out, buf, a_scr, sem):
    L = idxs_vmem.shape[0]

    for i in range(L):
        idx = feature_smem[i, 0]
        pltpu.make_async_copy(table_out.at[pl.ds(idx, 1), :],
                              buf.at[pl.ds(i, 1), :], sem).start()
    for i in range(L):
        idx = feature_smem[i, 0]
        pltpu.make_async_copy(table_out.at[pl.ds(idx, 1), :],
                              buf.at[pl.ds(i, 1), :], sem).wait()

    wih = wih_t[...].astype(jnp.bfloat16)
    whh = whh_t[...].astype(jnp.bfloat16)
    b = bias[...]
    idxs = idxs_vmem[...]
    a_scr[...] = jnp.dot(buf[...].astype(jnp.bfloat16), wih,
                         preferred_element_type=jnp.float32) + b

    h = h0_ref[...].astype(jnp.bfloat16)
    for i in range(L):
        pre = a_scr[i:i + 1, :] + jnp.dot(h, whh,
                                          preferred_element_type=jnp.float32)
        h_new = jnp.tanh(pre)
        dt = (feature_smem[i, 1]
              - feature_smem[(i - 1) % L, 1]).astype(jnp.float32)
        h = (h_new * (1.0 / dt + 1.0)).astype(jnp.bfloat16)
        mask = idxs == feature_smem[i, 0]
        buf[...] = jnp.where(mask, h_new, buf[...])
        a_new = jnp.dot(h_new.astype(jnp.bfloat16), wih,
                        preferred_element_type=jnp.float32) + b
        a_scr[...] = jnp.where(mask, a_new, a_scr[...])

    for i in range(L):
        idx = feature_smem[i, 0]
        pltpu.make_async_copy(buf.at[pl.ds(i, 1), :],
                              table_out.at[pl.ds(idx, 1), :], sem).start()
    for i in range(L):
        idx = feature_smem[i, 0]
        pltpu.make_async_copy(buf.at[pl.ds(i, 1), :],
                              table_out.at[pl.ds(idx, 1), :], sem).wait()


def kernel(feature, item_embedding, W_ih, W_hh, b_ih, b_hh, h0):
    L = feature.shape[0]
    M, D = item_embedding.shape
    H = W_ih.shape[0]
    wih_t = W_ih.T
    whh_t = W_hh.T
    bias = (b_ih + b_hh).reshape(1, H)
    idxs2d = feature[:, 0:1]
    h02d = h0.reshape(1, H)

    return pl.pallas_call(
        _rnn_update_kernel,
        out_shape=jax.ShapeDtypeStruct((M, D), item_embedding.dtype),
        in_specs=[
            pl.BlockSpec(memory_space=pltpu.MemorySpace.SMEM),
            pl.BlockSpec(memory_space=pltpu.MemorySpace.VMEM),
            pl.BlockSpec(memory_space=pltpu.MemorySpace.VMEM),
            pl.BlockSpec(memory_space=pltpu.MemorySpace.VMEM),
            pl.BlockSpec(memory_space=pltpu.MemorySpace.VMEM),
            pl.BlockSpec(memory_space=pltpu.MemorySpace.VMEM),
            pl.BlockSpec(memory_space=pltpu.MemorySpace.HBM),
        ],
        out_specs=pl.BlockSpec(memory_space=pltpu.MemorySpace.HBM),
        input_output_aliases={6: 0},
        scratch_shapes=[
            pltpu.VMEM((L, D), jnp.float32),
            pltpu.VMEM((L, D), jnp.float32),
            pltpu.SemaphoreType.DMA,
        ],
    )(feature, idxs2d, wih_t, whh_t, bias, h02d, item_embedding)


# speedup vs baseline: 1.0051x; 1.0051x over previous
import jax
import jax.numpy as jnp
from jax.experimental import pallas as pl
from jax.experimental.pallas import tpu as pltpu


def _rnn_update_kernel(feature_smem, idxs_vmem, wih_t, whh_t, bias, h0_ref,
                       table_in, table_out, buf, a_scr, sem):
    L = idxs_vmem.shape[0]

    for i in range(L):
        idx = feature_smem[i, 0]
        pltpu.make_async_copy(table_out.at[pl.ds(idx, 1), :],
                              buf.at[pl.ds(i, 1), :], sem).start()
    for i in range(L):
        idx = feature_smem[i, 0]
        pltpu.make_async_copy(table_out.at[pl.ds(idx, 1), :],
                              buf.at[pl.ds(i, 1), :], sem).wait()

    wih = wih_t[...]
    whh = whh_t[...]
    b = bias[...]
    idxs = idxs_vmem[...]
    a_scr[...] = jnp.dot(buf[...], wih, preferred_element_type=jnp.float32) + b

    h = h0_ref[...]
    for i in range(L):
        pre = a_scr[i:i + 1, :] + jnp.dot(h, whh,
                                          preferred_element_type=jnp.float32)
        h_new = jnp.tanh(pre)
        dt = (feature_smem[i, 1]
              - feature_smem[(i - 1) % L, 1]).astype(jnp.float32)
        h = h_new * (1.0 / dt + 1.0)
        mask = idxs == feature_smem[i, 0]
        buf[...] = jnp.where(mask, h_new, buf[...])
        a_new = jnp.dot(h_new, wih, preferred_element_type=jnp.float32) + b
        a_scr[...] = jnp.where(mask, a_new, a_scr[...])

    for i in range(L):
        idx = feature_smem[i, 0]
        pltpu.make_async_copy(buf.at[pl.ds(i, 1), :],
                              table_out.at[pl.ds(idx, 1), :], sem).start()
    for i in range(L):
        idx = feature_smem[i, 0]
        pltpu.make_async_copy(buf.at[pl.ds(i, 1), :],
                              table_out.at[pl.ds(idx, 1), :], sem).wait()


def kernel(feature, item_embedding, W_ih, W_hh, b_ih, b_hh, h0):
    L = feature.shape[0]
    M, D = item_embedding.shape
    H = W_ih.shape[0]
    wih_t = W_ih.T
    whh_t = W_hh.T
    bias = (b_ih + b_hh).reshape(1, H)
    idxs2d = feature[:, 0:1]
    h02d = h0.reshape(1, H)

    return pl.pallas_call(
        _rnn_update_kernel,
        out_shape=jax.ShapeDtypeStruct((M, D), item_embedding.dtype),
        in_specs=[
            pl.BlockSpec(memory_space=pltpu.MemorySpace.SMEM),
            pl.BlockSpec(memory_space=pltpu.MemorySpace.VMEM),
            pl.BlockSpec(memory_space=pltpu.MemorySpace.VMEM),
            pl.BlockSpec(memory_space=pltpu.MemorySpace.VMEM),
            pl.BlockSpec(memory_space=pltpu.MemorySpace.VMEM),
            pl.BlockSpec(memory_space=pltpu.MemorySpace.VMEM),
            pl.BlockSpec(memory_space=pltpu.MemorySpace.HBM),
        ],
        out_specs=pl.BlockSpec(memory_space=pltpu.MemorySpace.HBM),
        input_output_aliases={6: 0},
        scratch_shapes=[
            pltpu.VMEM((L, D), jnp.float32),
            pltpu.VMEM((L, D), jnp.float32),
            pltpu.SemaphoreType.DMA,
        ],
    )(feature, idxs2d, wih_t, whh_t, bias, h02d, item_embedding)
